# baseline (device time: 20571 ns/iter reference)
import jax
import jax.numpy as jnp
from jax import lax
from jax.experimental import pallas as pl
from jax.experimental.pallas import tpu as pltpu

BLOCK_M = 256
EPS = 1e-5


def kernel(x, dy, gamma):
    del gamma
    m, d = x.shape
    n_steps = m // BLOCK_M

    def body(x_ref, dy_ref, out_ref, comm_ref, send_sem, recv_sem):
        step = pl.program_id(0)
        my_x = lax.axis_index("x")
        my_y = lax.axis_index("y")
        my_z = lax.axis_index("z")
        peer = (1 - my_x, my_y, my_z)
        barrier_sem = pltpu.get_barrier_semaphore()

        @pl.when(step == 0)
        def _init():
            comm_ref[0] = jnp.zeros_like(comm_ref.at[0])
            pl.semaphore_signal(
                barrier_sem, inc=1,
                device_id=peer, device_id_type=pl.DeviceIdType.MESH,
            )

        xb = x_ref[:, :]
        dyb = dy_ref[:, :]
        s1 = jnp.sum(xb, axis=1, keepdims=True)
        s2 = jnp.sum(xb * xb, axis=1, keepdims=True)
        mu = s1 * (1.0 / d)
        var = s2 * (1.0 / d) - mu * mu
        rstd = lax.rsqrt(var + EPS)
        b = mu * rstd
        dgamma_b = jnp.sum(dyb * (xb * rstd - b), axis=0, keepdims=True)
        dbeta_b = jnp.sum(dyb, axis=0, keepdims=True)
        comm_ref[0, 0:1, :] += dgamma_b
        comm_ref[0, 1:2, :] += dbeta_b

        @pl.when(step == n_steps - 1)
        def _exchange():
            pl.semaphore_wait(barrier_sem, 1)
            rdma = pltpu.make_async_remote_copy(
                src_ref=comm_ref.at[0],
                dst_ref=comm_ref.at[1],
                send_sem=send_sem,
                recv_sem=recv_sem,
                device_id=peer,
                device_id_type=pl.DeviceIdType.MESH,
            )
            rdma.start()
            rdma.wait()
            out_ref[:, :] = comm_ref[0] + comm_ref[1]

    return pl.pallas_call(
        body,
        grid=(n_steps,),
        out_shape=jax.ShapeDtypeStruct((2, d), jnp.float32),
        in_specs=[
            pl.BlockSpec((BLOCK_M, d), lambda i: (i, 0)),
            pl.BlockSpec((BLOCK_M, d), lambda i: (i, 0)),
        ],
        out_specs=pl.BlockSpec((2, d), lambda i: (0, 0)),
        scratch_shapes=[
            pltpu.VMEM((2, 2, d), jnp.float32),
            pltpu.SemaphoreType.DMA,
            pltpu.SemaphoreType.DMA,
        ],
        compiler_params=pltpu.CompilerParams(
            collective_id=0,
            dimension_semantics=("arbitrary",),
        ),
    )(x, dy)


# device time: 19627 ns/iter; 1.0481x vs baseline; 1.0481x over previous
import jax
import jax.numpy as jnp
from jax import lax
from jax.experimental import pallas as pl
from jax.experimental.pallas import tpu as pltpu

BLOCK_M = 256
EPS = 1e-5


def kernel(x, dy, gamma):
    del gamma
    m, d = x.shape
    n_steps = (m // 2) // BLOCK_M

    def body(xa_ref, xb_ref, dya_ref, dyb_ref, out_ref, comm_ref,
             send_sem, recv_sem):
        step = pl.program_id(0)
        my_x = lax.axis_index("x")
        my_y = lax.axis_index("y")
        my_z = lax.axis_index("z")
        peer = (1 - my_x, my_y, my_z)
        barrier_sem = pltpu.get_barrier_semaphore()

        @pl.when(step == 0)
        def _init():
            comm_ref[0] = jnp.zeros_like(comm_ref.at[0])
            pl.semaphore_signal(
                barrier_sem, inc=1,
                device_id=peer, device_id_type=pl.DeviceIdType.MESH,
            )

        def block_partial(xb, dyb):
            s1 = jnp.sum(xb, axis=1, keepdims=True)
            s2 = jnp.sum(xb * xb, axis=1, keepdims=True)
            mu = s1 * (1.0 / d)
            var = s2 * (1.0 / d) - mu * mu
            rstd = lax.rsqrt(var + EPS)
            b = mu * rstd
            dg = jnp.sum(dyb * (xb * rstd - b), axis=0, keepdims=True)
            db = jnp.sum(dyb, axis=0, keepdims=True)
            return dg, db

        dg_a, db_a = block_partial(xa_ref[:, :], dya_ref[:, :])
        dg_b, db_b = block_partial(xb_ref[:, :], dyb_ref[:, :])
        comm_ref[0, 0:1, :] += dg_a + dg_b
        comm_ref[0, 1:2, :] += db_a + db_b

        @pl.when(step == n_steps - 1)
        def _exchange():
            pl.semaphore_wait(barrier_sem, 1)
            rdma = pltpu.make_async_remote_copy(
                src_ref=comm_ref.at[0],
                dst_ref=comm_ref.at[1],
                send_sem=send_sem,
                recv_sem=recv_sem,
                device_id=peer,
                device_id_type=pl.DeviceIdType.MESH,
            )
            rdma.start()
            rdma.wait()
            out_ref[:, :] = comm_ref[0] + comm_ref[1]

    half_blocks = n_steps
    return pl.pallas_call(
        body,
        grid=(n_steps,),
        out_shape=jax.ShapeDtypeStruct((2, d), jnp.float32),
        in_specs=[
            pl.BlockSpec((BLOCK_M, d), lambda i: (i, 0)),
            pl.BlockSpec((BLOCK_M, d), lambda i: (i + half_blocks, 0)),
            pl.BlockSpec((BLOCK_M, d), lambda i: (i, 0)),
            pl.BlockSpec((BLOCK_M, d), lambda i: (i + half_blocks, 0)),
        ],
        out_specs=pl.BlockSpec((2, d), lambda i: (0, 0)),
        scratch_shapes=[
            pltpu.VMEM((2, 2, d), jnp.float32),
            pltpu.SemaphoreType.DMA,
            pltpu.SemaphoreType.DMA,
        ],
        compiler_params=pltpu.CompilerParams(
            collective_id=0,
            dimension_semantics=("arbitrary",),
        ),
    )(x, x, dy, dy)


# device time: 15795 ns/iter; 1.3024x vs baseline; 1.2426x over previous
import jax
import jax.numpy as jnp
from jax import lax
from jax.experimental import pallas as pl
from jax.experimental.pallas import tpu as pltpu

BLOCK_M = 256
EPS = 1e-5
N_DEV = 8


def kernel(x, dy, gamma):
    del gamma
    m, d = x.shape
    rows_per_dev = m // 4
    n_steps = rows_per_dev // BLOCK_M
    blocks_per_q = n_steps

    def body(q_ref, x_ref, dy_ref, out_ref, comm_ref, send_sems, recv_sems):
        step = pl.program_id(0)
        my_x = lax.axis_index("x")
        my_y = lax.axis_index("y")
        my_z = lax.axis_index("z")
        g = 4 * my_x + 2 * my_y + my_z
        barrier_sem = pltpu.get_barrier_semaphore()

        @pl.when(step == 0)
        def _init():
            comm_ref[0] = jnp.zeros_like(comm_ref.at[0])
            for k in range(1, N_DEV):
                t = lax.rem(g + k, N_DEV)
                pl.semaphore_signal(
                    barrier_sem, inc=1,
                    device_id=(t // 4, lax.rem(t // 2, 2), lax.rem(t, 2)),
                    device_id_type=pl.DeviceIdType.MESH,
                )

        xb = x_ref[:, :]
        dyb = dy_ref[:, :]
        s1 = jnp.sum(xb, axis=1, keepdims=True)
        s2 = jnp.sum(xb * xb, axis=1, keepdims=True)
        mu = s1 * (1.0 / d)
        var = s2 * (1.0 / d) - mu * mu
        rstd = lax.rsqrt(var + EPS)
        b = mu * rstd
        dgamma_b = jnp.sum(dyb * (xb * rstd - b), axis=0, keepdims=True)
        dbeta_b = jnp.sum(dyb, axis=0, keepdims=True)
        comm_ref[0, 0:1, :] += dgamma_b
        comm_ref[0, 1:2, :] += dbeta_b

        @pl.when(step == n_steps - 1)
        def _exchange():
            pl.semaphore_wait(barrier_sem, N_DEV - 1)
            rdmas = []
            for k in range(1, N_DEV):
                t = lax.rem(g + k, N_DEV)
                rdma = pltpu.make_async_remote_copy(
                    src_ref=comm_ref.at[0],
                    dst_ref=comm_ref.at[k],
                    send_sem=send_sems.at[k],
                    recv_sem=recv_sems.at[k],
                    device_id=(t // 4, lax.rem(t // 2, 2), lax.rem(t, 2)),
                    device_id_type=pl.DeviceIdType.MESH,
                )
                rdma.start()
                rdmas.append(rdma)
            for rdma in rdmas:
                rdma.wait()
            out_ref[:, :] = jnp.sum(comm_ref[:, :, :], axis=0)

    grid_spec = pltpu.PrefetchScalarGridSpec(
        num_scalar_prefetch=1,
        grid=(n_steps,),
        in_specs=[
            pl.BlockSpec((BLOCK_M, d), lambda i, q: (q[0] * blocks_per_q + i, 0)),
            pl.BlockSpec((BLOCK_M, d), lambda i, q: (q[0] * blocks_per_q + i, 0)),
        ],
        out_specs=pl.BlockSpec((2, d), lambda i, q: (0, 0)),
        scratch_shapes=[
            pltpu.VMEM((N_DEV, 2, d), jnp.float32),
            pltpu.SemaphoreType.DMA((N_DEV,)),
            pltpu.SemaphoreType.DMA((N_DEV,)),
        ],
    )

    q = (2 * lax.axis_index("y") + lax.axis_index("z")).astype(jnp.int32)
    return pl.pallas_call(
        body,
        grid_spec=grid_spec,
        out_shape=jax.ShapeDtypeStruct((2, d), jnp.float32),
        compiler_params=pltpu.CompilerParams(
            collective_id=0,
            dimension_semantics=("arbitrary",),
        ),
    )(jnp.reshape(q, (1,)), x, dy)
